# baseline (speedup 1.0000x reference)
import functools

import jax
import jax.numpy as jnp
from jax import lax
from jax.experimental import pallas as pl
from jax.experimental.pallas import tpu as pltpu
from jax.experimental.pallas import tpu_sc as plsc

DIM = 64
PDIM = 128
NW = 32
SEG = 128
CHUNK = 256
STREAMS = CHUNK // SEG


def _make_gather(n_idx):
    per_w = n_idx // NW
    seg_per_w = per_w // SEG
    n_chunk = per_w // CHUNK
    assert n_chunk % 2 == 0 and n_chunk >= 4
    mesh = plsc.VectorSubcoreMesh(core_axis_name="c", subcore_axis_name="s")

    @functools.partial(
        pl.kernel,
        mesh=mesh,
        out_type=jax.ShapeDtypeStruct((n_idx, PDIM), jnp.float32),
        scratch_types=[
            pltpu.VMEM((seg_per_w, SEG), jnp.int32),
            pltpu.VMEM((2, CHUNK, PDIM), jnp.float32),
            pltpu.SemaphoreType.DMA,
            pltpu.SemaphoreType.DMA,
            pltpu.SemaphoreType.DMA,
            pltpu.SemaphoreType.DMA,
        ],
        compiler_params=pltpu.CompilerParams(use_tc_tiling_on_sc=False),
    )
    def gather(idx_hbm, table_hbm, out_hbm, idx_v, rows_v, g0, g1, s0, s1):
        gsem = (g0, g1)
        ssem = (s0, s1)
        wid = lax.axis_index("s") * 2 + lax.axis_index("c")
        base = wid * per_w

        pltpu.sync_copy(idx_hbm.at[pl.ds(wid * seg_per_w, seg_per_w)], idx_v)

        def fire_gathers(g, bf):
            for j in range(STREAMS):
                pltpu.async_copy(
                    table_hbm.at[idx_v.at[g * STREAMS + j]],
                    rows_v.at[bf, pl.ds(j * SEG, SEG)],
                    gsem[bf],
                )

        def wait_gathers(bf):
            pltpu.make_async_copy(
                table_hbm.at[pl.ds(0, CHUNK)], rows_v.at[bf], gsem[bf]
            ).wait()

        def fire_store(g, bf):
            pltpu.async_copy(
                rows_v.at[bf], out_hbm.at[pl.ds(base + g * CHUNK, CHUNK)],
                ssem[bf],
            )

        def wait_store(bf):
            pltpu.make_async_copy(
                rows_v.at[bf], out_hbm.at[pl.ds(base, CHUNK)], ssem[bf]
            ).wait()

        fire_gathers(0, 0)
        fire_gathers(1, 1)
        wait_gathers(0)
        fire_store(0, 0)

        def body(p, _):
            g = 2 * p + 2
            for bf in (0, 1):
                wait_gathers(1 - bf)
                fire_store(g - 1, 1 - bf)
                wait_store(bf)
                fire_gathers(g, bf)
                g = g + 1
            return _

        lax.fori_loop(0, (n_chunk - 2) // 2, body, None)

        wait_gathers(1)
        fire_store(n_chunk - 1, 1)
        wait_store(0)
        wait_store(1)

    return gather


_PADBLK = 512


def _pad_table(table_t):
    v = table_t.shape[1]

    def body(x_ref, o_ref):
        x = x_ref[...]
        eye = (jax.lax.broadcasted_iota(jnp.int32, (DIM, PDIM), 0)
               == jax.lax.broadcasted_iota(jnp.int32, (DIM, PDIM), 1)
               ).astype(jnp.float32)
        o_ref[...] = jax.lax.dot_general(
            x, eye, (((0,), (0,)), ((), ())),
            preferred_element_type=jnp.float32)

    return pl.pallas_call(
        body,
        grid=(pl.cdiv(v, _PADBLK),),
        in_specs=[pl.BlockSpec((DIM, _PADBLK), lambda i: (0, i))],
        out_specs=pl.BlockSpec((_PADBLK, PDIM), lambda i: (i, 0)),
        out_shape=jax.ShapeDtypeStruct((v, PDIM), jnp.float32),
    )(table_t)


def kernel(token_id, table):
    b, l = token_id.shape
    v, d = table.shape
    n = b * l
    idx2d = token_id.reshape(n // SEG, SEG).astype(jnp.int32)
    table_p = _pad_table(table.T)
    out = _make_gather(n)(idx2d, table_p)
    return out[:, :DIM].reshape(b, l, DIM)


# speedup vs baseline: 2.3407x; 2.3407x over previous
import functools

import jax
import jax.numpy as jnp
from jax import lax
from jax.experimental import pallas as pl
from jax.experimental.pallas import tpu as pltpu
from jax.experimental.pallas import tpu_sc as plsc

DIM = 64
PDIM = 128
NW = 32
SEG = 128
CHUNK = 256
STREAMS = CHUNK // SEG


def _make_gather(n_idx):
    per_w = n_idx // NW
    seg_per_w = per_w // SEG
    n_chunk = per_w // CHUNK
    assert n_chunk % 2 == 0 and n_chunk >= 4
    mesh = plsc.VectorSubcoreMesh(core_axis_name="c", subcore_axis_name="s")

    @functools.partial(
        pl.kernel,
        mesh=mesh,
        out_type=jax.ShapeDtypeStruct((n_idx, PDIM), jnp.float32),
        scratch_types=[
            pltpu.VMEM((seg_per_w, SEG), jnp.int32),
            pltpu.VMEM((2, CHUNK, PDIM), jnp.float32),
            pltpu.SemaphoreType.DMA,
            pltpu.SemaphoreType.DMA,
            pltpu.SemaphoreType.DMA,
            pltpu.SemaphoreType.DMA,
        ],
        compiler_params=pltpu.CompilerParams(use_tc_tiling_on_sc=False),
    )
    def gather(idx_hbm, table_hbm, out_hbm, idx_v, rows_v, g0, g1, s0, s1):
        gsem = (g0, g1)
        ssem = (s0, s1)
        wid = lax.axis_index("s") * 2 + lax.axis_index("c")
        base = wid * per_w

        pltpu.sync_copy(idx_hbm.at[pl.ds(wid * seg_per_w, seg_per_w)], idx_v)

        def fire_gathers(g, bf):
            for j in range(STREAMS):
                pltpu.async_copy(
                    table_hbm.at[idx_v.at[g * STREAMS + j]],
                    rows_v.at[bf, pl.ds(j * SEG, SEG)],
                    gsem[bf],
                )

        def wait_gathers(bf):
            pltpu.make_async_copy(
                table_hbm.at[pl.ds(0, CHUNK)], rows_v.at[bf], gsem[bf]
            ).wait()

        def fire_store(g, bf):
            pltpu.async_copy(
                rows_v.at[bf], out_hbm.at[pl.ds(base + g * CHUNK, CHUNK)],
                ssem[bf],
            )

        def wait_store(bf):
            pltpu.make_async_copy(
                rows_v.at[bf], out_hbm.at[pl.ds(base, CHUNK)], ssem[bf]
            ).wait()

        fire_gathers(0, 0)
        fire_gathers(1, 1)
        wait_gathers(0)
        fire_store(0, 0)

        def body(p, _):
            g = 2 * p + 2
            for bf in (0, 1):
                wait_gathers(1 - bf)
                fire_store(g - 1, 1 - bf)
                wait_store(bf)
                fire_gathers(g, bf)
                g = g + 1
            return _

        lax.fori_loop(0, (n_chunk - 2) // 2, body, None)

        wait_gathers(1)
        fire_store(n_chunk - 1, 1)
        wait_store(0)
        wait_store(1)

    return gather


_PADBLK = 8192


def _pad_table(table_t):
    v = table_t.shape[1]

    def body(x_ref, o_ref):
        x = x_ref[...]
        xt = jnp.transpose(x)
        o_ref[...] = jnp.concatenate(
            [xt, jnp.zeros((_PADBLK, PDIM - DIM), jnp.float32)], axis=1)

    return pl.pallas_call(
        body,
        grid=(pl.cdiv(v, _PADBLK),),
        in_specs=[pl.BlockSpec((DIM, _PADBLK), lambda i: (0, i))],
        out_specs=pl.BlockSpec((_PADBLK, PDIM), lambda i: (i, 0)),
        out_shape=jax.ShapeDtypeStruct((v, PDIM), jnp.float32),
    )(table_t)


def kernel(token_id, table):
    b, l = token_id.shape
    v, d = table.shape
    n = b * l
    idx2d = token_id.reshape(n // SEG, SEG).astype(jnp.int32)
    table_p = _pad_table(table.T)
    out = _make_gather(n)(idx2d, table_p)
    return out[:, :DIM].reshape(b, l, DIM)


# speedup vs baseline: 2.3937x; 1.0227x over previous
import functools

import jax
import jax.numpy as jnp
from jax import lax
from jax.experimental import pallas as pl
from jax.experimental.pallas import tpu as pltpu
from jax.experimental.pallas import tpu_sc as plsc

DIM = 64
PDIM = 128
NW = 32
SEG = 128
CHUNK = 256
STREAMS = CHUNK // SEG


def _make_gather(n_idx):
    per_w = n_idx // NW
    seg_per_w = per_w // SEG
    n_chunk = per_w // CHUNK
    assert n_chunk % 2 == 0 and n_chunk >= 4
    mesh = plsc.VectorSubcoreMesh(core_axis_name="c", subcore_axis_name="s")

    @functools.partial(
        pl.kernel,
        mesh=mesh,
        out_type=jax.ShapeDtypeStruct((n_idx, PDIM), jnp.float32),
        scratch_types=[
            pltpu.VMEM((seg_per_w, SEG), jnp.int32),
            pltpu.VMEM((2, CHUNK, PDIM), jnp.float32),
            pltpu.SemaphoreType.DMA,
            pltpu.SemaphoreType.DMA,
            pltpu.SemaphoreType.DMA,
            pltpu.SemaphoreType.DMA,
        ],
        compiler_params=pltpu.CompilerParams(use_tc_tiling_on_sc=False),
    )
    def gather(idx_hbm, table_hbm, out_hbm, idx_v, rows_v, g0, g1, s0, s1):
        gsem = (g0, g1)
        ssem = (s0, s1)
        wid = lax.axis_index("s") * 2 + lax.axis_index("c")
        base = wid * per_w

        pltpu.sync_copy(idx_hbm.at[pl.ds(wid * seg_per_w, seg_per_w)], idx_v)

        def fire_gathers(g, bf):
            for j in range(STREAMS):
                pltpu.async_copy(
                    table_hbm.at[idx_v.at[g * STREAMS + j]],
                    rows_v.at[bf, pl.ds(j * SEG, SEG)],
                    gsem[bf],
                )

        def wait_gathers(bf):
            pltpu.make_async_copy(
                table_hbm.at[pl.ds(0, CHUNK)], rows_v.at[bf], gsem[bf]
            ).wait()

        def fire_store(g, bf):
            pltpu.async_copy(
                rows_v.at[bf], out_hbm.at[pl.ds(base + g * CHUNK, CHUNK)],
                ssem[bf],
            )

        def wait_store(bf):
            pltpu.make_async_copy(
                rows_v.at[bf], out_hbm.at[pl.ds(base, CHUNK)], ssem[bf]
            ).wait()

        fire_gathers(0, 0)
        fire_gathers(1, 1)
        wait_gathers(0)
        fire_store(0, 0)

        def body(p, _):
            g = 2 * p + 2
            for bf in (0, 1):
                wait_gathers(1 - bf)
                fire_store(g - 1, 1 - bf)
                wait_store(bf)
                fire_gathers(g, bf)
                g = g + 1
            return _

        lax.fori_loop(0, (n_chunk - 2) // 2, body, None)

        wait_gathers(1)
        fire_store(n_chunk - 1, 1)
        wait_store(0)
        wait_store(1)

    return gather


_PADBLK = 16384


def _pad_table(table_t):
    v = table_t.shape[1]

    def body(x_ref, o_ref):
        x = x_ref[...]
        xt = jnp.transpose(x)
        o_ref[...] = jnp.concatenate(
            [xt, jnp.zeros((_PADBLK, PDIM - DIM), jnp.float32)], axis=1)

    return pl.pallas_call(
        body,
        grid=(pl.cdiv(v, _PADBLK),),
        in_specs=[pl.BlockSpec((DIM, _PADBLK), lambda i: (0, i))],
        out_specs=pl.BlockSpec((_PADBLK, PDIM), lambda i: (i, 0)),
        out_shape=jax.ShapeDtypeStruct((v, PDIM), jnp.float32),
    )(table_t)


def kernel(token_id, table):
    b, l = token_id.shape
    v, d = table.shape
    n = b * l
    idx2d = token_id.reshape(n // SEG, SEG).astype(jnp.int32)
    table_p = _pad_table(table.T)
    out = _make_gather(n)(idx2d, table_p)
    return out[:, :DIM].reshape(b, l, DIM)
